# baseline (speedup 1.0000x reference)
import functools

import jax
import jax.numpy as jnp
from jax import lax
from jax.experimental import pallas as pl
from jax.experimental.pallas import tpu as pltpu
from jax.experimental.pallas import tpu_sc as plsc

_NUM_POINT = 128
_INIT_STRIDE = 10.0
_COARSE_STRIDE = 4.0
_DOWN_SAMPLE = 4.0
_TH = 8

_N = 400
_P = 129
_NPTS = _N * _P
_NW = 32
_CW = 128
_NCHUNK = 13
_PPW = _CW * _NCHUNK
_NPID = _NW * _PPW
_NBLK = 16


def _conv_body(x_ref, w1_ref, b1_ref, w2_ref, b2_ref, out_ref, xt_s):
    w1 = w1_ref[...]
    w2 = w2_ref[...]
    b1 = b1_ref[...]
    b2 = b2_ref[...]
    xt_s[0:1, :, :] = jnp.zeros((1, 130, 64), jnp.float32)
    xt_s[129:130, :, :] = jnp.zeros((1, 130, 64), jnp.float32)
    xt_s[1:129, 0:1, :] = jnp.zeros((128, 1, 64), jnp.float32)
    xt_s[1:129, 129:130, :] = jnp.zeros((128, 1, 64), jnp.float32)
    for r0 in range(0, 128, 16):
        blk = x_ref[0][:, r0:r0 + 16, :].reshape(64, 16 * 128)
        xt_s[r0 + 1:r0 + 17, 1:129, :] = blk.T.reshape(16, 128, 64)
    for h0 in range(0, 128, _TH):
        pieces = []
        for dy in range(3):
            for dx in range(3):
                xs = xt_s[h0 + dy:h0 + dy + _TH, dx:dx + 128, :]
                pieces.append(xs.reshape(_TH * 128, 64))
        x9 = jnp.concatenate(pieces, axis=1)
        acc = jnp.dot(x9, w1, preferred_element_type=jnp.float32) + b1
        acc = jnp.maximum(acc, 0.0)
        z = jnp.dot(acc, w2, preferred_element_type=jnp.float32) + b2
        out_ref[0, h0:h0 + _TH, :, :] = z.reshape(_TH, 128, 64)


def _fused_conv(x_nchw, conv1_w, conv1_b, conv2_w, conv2_b):
    B = x_nchw.shape[0]
    w1 = conv1_w.transpose(2, 3, 1, 0).reshape(576, 256)
    w2 = conv2_w[:, :, 0, 0].T
    return pl.pallas_call(
        _conv_body,
        grid=(B,),
        in_specs=[
            pl.BlockSpec((1, 64, 128, 128), lambda b: (b, 0, 0, 0)),
            pl.BlockSpec((576, 256), lambda b: (0, 0)),
            pl.BlockSpec((1, 256), lambda b: (0, 0)),
            pl.BlockSpec((256, 64), lambda b: (0, 0)),
            pl.BlockSpec((1, 64), lambda b: (0, 0)),
        ],
        out_specs=pl.BlockSpec((1, 128, 128, 64), lambda b: (b, 0, 0, 0)),
        out_shape=jax.ShapeDtypeStruct((B, 128, 128, 64), jnp.float32),
        scratch_shapes=[pltpu.VMEM((130, 130, 64), jnp.float32)],
    )(x_nchw, w1, conv1_b.reshape(1, 256), w2, conv2_b.reshape(1, 64))


def _whg_body(wh_ref, flat_ref, img_ref, out_ref):
    b = pl.program_id(0)
    j = pl.program_id(1)

    @pl.when((b == 0) & (j == 0))
    def _():
        out_ref[...] = jnp.zeros((256, _N), jnp.float32)

    whb = wh_ref[0].reshape(256, 16 * 128)
    rowflat = (jax.lax.broadcasted_iota(jnp.int32, (16 * 128, 1), 0)
               + j * (16 * 128))
    onehot = ((rowflat == flat_ref[...]) &
              (img_ref[...] == b)).astype(jnp.float32)
    out_ref[...] += jnp.dot(whb, onehot, preferred_element_type=jnp.float32)


def _wh_gather(wh, flat_n, img_n):
    return pl.pallas_call(
        _whg_body,
        grid=(4, 8),
        in_specs=[
            pl.BlockSpec((1, 256, 16, 128), lambda b, j: (b, 0, j, 0)),
            pl.BlockSpec((1, _N), lambda b, j: (0, 0)),
            pl.BlockSpec((1, _N), lambda b, j: (0, 0)),
        ],
        out_specs=pl.BlockSpec((256, _N), lambda b, j: (0, 0)),
        out_shape=jax.ShapeDtypeStruct((256, _N), jnp.float32),
    )(wh, flat_n.reshape(1, _N), img_n.reshape(1, _N))


def _prep_body(pts_ref, idx_ref, wgt_ref):
    x = pts_ref[0:1, :] - 0.5
    y = pts_ref[1:2, :] - 0.5
    imgi = pts_ref[2:3, :].astype(jnp.int32)
    x0 = jnp.floor(x)
    y0 = jnp.floor(y)
    wx1 = x - x0
    wy1 = y - y0
    for k, (dy, dx) in enumerate(((0, 0), (0, 1), (1, 0), (1, 1))):
        xc = x0 + dx
        yc = y0 + dy
        valid = (xc >= 0.0) & (xc < 128.0) & (yc >= 0.0) & (yc < 128.0)
        xi = jnp.clip(xc, 0.0, 127.0).astype(jnp.int32)
        yi = jnp.clip(yc, 0.0, 127.0).astype(jnp.int32)
        flat = imgi * 16384 + yi * 128 + xi
        idx = lax.shift_right_logical(flat, 1)
        odd = (flat & 1).astype(jnp.float32)
        wk = ((wx1 if dx else 1.0 - wx1) * (wy1 if dy else 1.0 - wy1)
              * valid.astype(jnp.float32))
        idx_ref[k:k + 1, :] = idx
        rows = _NBLK * _P
        for j in range(_N // _NBLK):
            sl = slice(j * rows, (j + 1) * rows)
            wgt_ref[j, 2 * k:2 * k + 1, :] = (wk * (1.0 - odd))[0:1, sl]
            wgt_ref[j, 2 * k + 1:2 * k + 2, :] = (wk * odd)[0:1, sl]


def _prep(pts3):
    return pl.pallas_call(
        _prep_body,
        out_shape=(jax.ShapeDtypeStruct((4, _NPID), jnp.int32),
                   jax.ShapeDtypeStruct((_N // _NBLK, 8, _NBLK * _P),
                                        jnp.float32)),
    )(pts3)


_NSLOT = 4
_TOTCH = 4 * _NCHUNK


def _sc_gather_body(table, idx_hbm, out, idx_v,
                    b0, b1, b2, b3, g0, g1, g2, g3, w0, w1, w2, w3):
    wid = lax.axis_index("c") * 16 + lax.axis_index("s")
    base = wid * _PPW
    pltpu.sync_copy(idx_hbm.at[:, wid], idx_v)
    bufs = (b0, b1, b2, b3)
    gsems = (g0, g1, g2, g3)
    wsems = (w0, w1, w2, w3)

    def gather(i, b, gsem):
        k = i // _NCHUNK
        c = i - k * _NCHUNK
        return pltpu.make_async_copy(table.at[idx_v.at[k, c]], b, gsem)

    def write(i, b, wsem):
        k = i // _NCHUNK
        c = i - k * _NCHUNK
        return pltpu.make_async_copy(
            b, out.at[k, pl.ds(base + c * _CW, _CW)], wsem)

    for b in range(_NSLOT):
        gather(b, bufs[b], gsems[b]).start()

    def body(j, carry):
        for b in range(_NSLOT):
            i = j * _NSLOT + b
            gather(i, bufs[b], gsems[b]).wait()
            write(i, bufs[b], wsems[b]).start()
            i2 = i + _NSLOT

            @pl.when(i2 < _TOTCH)
            def _():
                write(i, bufs[b], wsems[b]).wait()
                gather(i2, bufs[b], gsems[b]).start()
        return carry

    lax.fori_loop(0, _TOTCH // _NSLOT, body, 0)
    for b in range(_NSLOT):
        write(_TOTCH - _NSLOT + b, bufs[b], wsems[b]).wait()


@functools.lru_cache(maxsize=1)
def _sc_gather_fn():
    mesh = plsc.VectorSubcoreMesh(core_axis_name="c", subcore_axis_name="s")
    return functools.partial(
        pl.kernel, mesh=mesh,
        out_type=jax.ShapeDtypeStruct((4, _NPID, 128), jnp.float32),
        scratch_types=[
            pltpu.VMEM((4, _NCHUNK, _CW), jnp.int32),
            pltpu.VMEM((_CW, 128), jnp.float32),
            pltpu.VMEM((_CW, 128), jnp.float32),
            pltpu.VMEM((_CW, 128), jnp.float32),
            pltpu.VMEM((_CW, 128), jnp.float32),
            pltpu.SemaphoreType.DMA,
            pltpu.SemaphoreType.DMA,
            pltpu.SemaphoreType.DMA,
            pltpu.SemaphoreType.DMA,
            pltpu.SemaphoreType.DMA,
            pltpu.SemaphoreType.DMA,
            pltpu.SemaphoreType.DMA,
            pltpu.SemaphoreType.DMA,
        ],
    )(_sc_gather_body)


def _gather_corners(table, idx4):
    return _sc_gather_fn()(table, idx4)


def _combine_body(g_ref, w_ref, fp_ref):
    w = w_ref[0]
    acc = None
    for k in range(4):
        gk = g_ref[k]
        term = (gk[:, 0:64] * w[2 * k][:, None]
                + gk[:, 64:128] * w[2 * k + 1][:, None])
        acc = term if acc is None else acc + term
    fp_ref[...] = acc


def _combine(g, wgt4):
    rows = _NBLK * _P
    return pl.pallas_call(
        _combine_body,
        grid=(_N // _NBLK,),
        in_specs=[
            pl.BlockSpec((4, rows, 128), lambda i: (0, i, 0)),
            pl.BlockSpec((1, 8, rows), lambda i: (i, 0, 0)),
        ],
        out_specs=pl.BlockSpec((rows, 64), lambda i: (i, 0)),
        out_shape=jax.ShapeDtypeStruct((_NPTS, 64), jnp.float32),
    )(g, wgt4)


def _wcomb_body(fw_ref, pw_ref, out_ref):
    pw = pw_ref[...]
    for o0 in range(0, 256, 64):
        wc = jnp.dot(fw_ref[o0:o0 + 64, :], pw,
                     preferred_element_type=jnp.float32)
        t = wc.reshape(64, 64, _P).transpose(0, 2, 1)
        out_ref[o0:o0 + 64] = t


def _wcomb(fuse_w, poly_w):
    return pl.pallas_call(
        _wcomb_body,
        out_shape=jax.ShapeDtypeStruct((256, _P, 64), jnp.float32),
    )(fuse_w, poly_w)


def _tail_body(fp_ref, ip_ref, wc_ref, fb_ref, pi_ref, pc_ref):
    fp = fp_ref[...]
    offs = jax.lax.dot_general(fp, wc_ref[...], (((1,), (1,)), ((), ())),
                               preferred_element_type=jnp.float32)
    offs = offs + fb_ref[...]
    ip = ip_ref[...]
    pi_ref[...] = ip * _DOWN_SAMPLE
    pc_ref[...] = offs * (_COARSE_STRIDE * _DOWN_SAMPLE) + ip * _DOWN_SAMPLE


def kernel(ct_hm, wh, cnn_feature, ct_01, ct_ind, ct_img_idx,
           conv1_w, conv1_b, conv2_w, conv2_b, poly_w, fuse_w, fuse_b):
    B, _, H, W = ct_hm.shape
    mask = ct_01.reshape(-1)
    ind = jnp.where(mask, ct_ind.reshape(-1), 0).astype(jnp.int32)
    img = jnp.where(mask, ct_img_idx.reshape(-1), 0).astype(jnp.int32)
    N = mask.shape[0]
    ct_x = ind % W
    ct_y = ind // W
    ct_offset = _wh_gather(wh, ind, img).T.reshape(N, -1, 2)
    ct = jnp.stack([ct_x.astype(jnp.float32), ct_y.astype(jnp.float32)], axis=1)
    init_polys = ct_offset * _INIT_STRIDE + ct[:, None, :]

    feat = _fused_conv(cnn_feature, conv1_w, conv1_b, conv2_w, conv2_b)
    table = feat.reshape(B * H * W // 2, 128)

    points = jnp.concatenate([ct[:, None, :], init_polys], axis=1)
    px = points[..., 0].reshape(-1)
    py = points[..., 1].reshape(-1)
    imgf = jnp.repeat(img.astype(jnp.float32), _P)
    pad = _NPID - _NPTS
    pts3 = jnp.stack([
        jnp.pad(px, (0, pad), constant_values=-1000.0),
        jnp.pad(py, (0, pad), constant_values=-1000.0),
        jnp.pad(imgf, (0, pad)),
    ] + [jnp.zeros(_NPID, jnp.float32)] * 5)

    idx4, wgt4 = _prep(pts3)
    idx4 = idx4.reshape(4, _NW, _NCHUNK, _CW)
    g = _gather_corners(table, idx4)
    fp = _combine(g, wgt4).reshape(_N, _P * 64)
    wcp = _wcomb(fuse_w, poly_w).reshape(256, _P * 64)

    ip_flat = init_polys.reshape(N, _NUM_POINT * 2)
    pi, pc = pl.pallas_call(
        _tail_body,
        out_shape=(jax.ShapeDtypeStruct((N, _NUM_POINT * 2), jnp.float32),
                   jax.ShapeDtypeStruct((N, _NUM_POINT * 2), jnp.float32)),
    )(fp, ip_flat, wcp, fuse_b.reshape(1, -1))
    return (pi.reshape(N, _NUM_POINT, 2), pc.reshape(N, _NUM_POINT, 2))


# speedup vs baseline: 1.0010x; 1.0010x over previous
import functools

import jax
import jax.numpy as jnp
from jax import lax
from jax.experimental import pallas as pl
from jax.experimental.pallas import tpu as pltpu
from jax.experimental.pallas import tpu_sc as plsc

_NUM_POINT = 128
_INIT_STRIDE = 10.0
_COARSE_STRIDE = 4.0
_DOWN_SAMPLE = 4.0
_TH = 8

_N = 400
_P = 129
_NPTS = _N * _P
_NW = 32
_CW = 128
_NCHUNK = 13
_PPW = _CW * _NCHUNK
_NPID = _NW * _PPW
_NBLK = 16


def _conv_body(x_ref, w1_ref, b1_ref, w2_ref, b2_ref, out_ref, xt_s):
    w1 = w1_ref[...]
    w2 = w2_ref[...]
    b1 = b1_ref[...]
    b2 = b2_ref[...]
    xt_s[0:1, :, :] = jnp.zeros((1, 130, 64), jnp.float32)
    xt_s[129:130, :, :] = jnp.zeros((1, 130, 64), jnp.float32)
    xt_s[1:129, 0:1, :] = jnp.zeros((128, 1, 64), jnp.float32)
    xt_s[1:129, 129:130, :] = jnp.zeros((128, 1, 64), jnp.float32)
    for r0 in range(0, 128, 16):
        blk = x_ref[0][:, r0:r0 + 16, :].reshape(64, 16 * 128)
        xt_s[r0 + 1:r0 + 17, 1:129, :] = blk.T.reshape(16, 128, 64)
    for h0 in range(0, 128, _TH):
        pieces = []
        for dy in range(3):
            for dx in range(3):
                xs = xt_s[h0 + dy:h0 + dy + _TH, dx:dx + 128, :]
                pieces.append(xs.reshape(_TH * 128, 64))
        x9 = jnp.concatenate(pieces, axis=1)
        acc = jnp.dot(x9, w1, preferred_element_type=jnp.float32) + b1
        acc = jnp.maximum(acc, 0.0)
        z = jnp.dot(acc, w2, preferred_element_type=jnp.float32) + b2
        out_ref[0, h0:h0 + _TH, :, :] = z.reshape(_TH, 128, 64)


def _fused_conv(x_nchw, conv1_w, conv1_b, conv2_w, conv2_b):
    B = x_nchw.shape[0]
    w1 = conv1_w.transpose(2, 3, 1, 0).reshape(576, 256)
    w2 = conv2_w[:, :, 0, 0].T
    return pl.pallas_call(
        _conv_body,
        grid=(B,),
        in_specs=[
            pl.BlockSpec((1, 64, 128, 128), lambda b: (b, 0, 0, 0)),
            pl.BlockSpec((576, 256), lambda b: (0, 0)),
            pl.BlockSpec((1, 256), lambda b: (0, 0)),
            pl.BlockSpec((256, 64), lambda b: (0, 0)),
            pl.BlockSpec((1, 64), lambda b: (0, 0)),
        ],
        out_specs=pl.BlockSpec((1, 128, 128, 64), lambda b: (b, 0, 0, 0)),
        out_shape=jax.ShapeDtypeStruct((B, 128, 128, 64), jnp.float32),
        scratch_shapes=[pltpu.VMEM((130, 130, 64), jnp.float32)],
    )(x_nchw, w1, conv1_b.reshape(1, 256), w2, conv2_b.reshape(1, 64))


def _whg_body(wh_ref, flat_ref, img_ref, out_ref):
    b = pl.program_id(0)
    j = pl.program_id(1)

    @pl.when((b == 0) & (j == 0))
    def _():
        out_ref[...] = jnp.zeros((256, _N), jnp.float32)

    whb = wh_ref[0].reshape(256, 16 * 128)
    rowflat = (jax.lax.broadcasted_iota(jnp.int32, (16 * 128, 1), 0)
               + j * (16 * 128))
    onehot = ((rowflat == flat_ref[...]) &
              (img_ref[...] == b)).astype(jnp.float32)
    out_ref[...] += jnp.dot(whb, onehot, preferred_element_type=jnp.float32)


def _wh_gather(wh, flat_n, img_n):
    return pl.pallas_call(
        _whg_body,
        grid=(4, 8),
        in_specs=[
            pl.BlockSpec((1, 256, 16, 128), lambda b, j: (b, 0, j, 0)),
            pl.BlockSpec((1, _N), lambda b, j: (0, 0)),
            pl.BlockSpec((1, _N), lambda b, j: (0, 0)),
        ],
        out_specs=pl.BlockSpec((256, _N), lambda b, j: (0, 0)),
        out_shape=jax.ShapeDtypeStruct((256, _N), jnp.float32),
    )(wh, flat_n.reshape(1, _N), img_n.reshape(1, _N))


def _prep_body(pts_ref, idx_ref, wgt_ref):
    x = pts_ref[0:1, :] - 0.5
    y = pts_ref[1:2, :] - 0.5
    imgi = pts_ref[2:3, :].astype(jnp.int32)
    x0 = jnp.floor(x)
    y0 = jnp.floor(y)
    wx1 = x - x0
    wy1 = y - y0
    for k, (dy, dx) in enumerate(((0, 0), (0, 1), (1, 0), (1, 1))):
        xc = x0 + dx
        yc = y0 + dy
        valid = (xc >= 0.0) & (xc < 128.0) & (yc >= 0.0) & (yc < 128.0)
        xi = jnp.clip(xc, 0.0, 127.0).astype(jnp.int32)
        yi = jnp.clip(yc, 0.0, 127.0).astype(jnp.int32)
        flat = imgi * 16384 + yi * 128 + xi
        idx = lax.shift_right_logical(flat, 1)
        odd = (flat & 1).astype(jnp.float32)
        wk = ((wx1 if dx else 1.0 - wx1) * (wy1 if dy else 1.0 - wy1)
              * valid.astype(jnp.float32))
        idx_ref[k:k + 1, :] = idx
        rows = _NBLK * _P
        for j in range(_N // _NBLK):
            sl = slice(j * rows, (j + 1) * rows)
            wgt_ref[j, 2 * k:2 * k + 1, :] = (wk * (1.0 - odd))[0:1, sl]
            wgt_ref[j, 2 * k + 1:2 * k + 2, :] = (wk * odd)[0:1, sl]


def _prep(pts3):
    return pl.pallas_call(
        _prep_body,
        out_shape=(jax.ShapeDtypeStruct((4, _NPID), jnp.int32),
                   jax.ShapeDtypeStruct((_N // _NBLK, 8, _NBLK * _P),
                                        jnp.float32)),
    )(pts3)


_NSLOT = 4
_C0N = 20
_C1N = 6


def _sc_gather_body(table, idx_hbm, out, idx_v,
                    b0, b1, b2, b3, g0, g1, g2, g3, w0, w1, w2, w3):
    cid = lax.axis_index("c")
    sid = lax.axis_index("s")
    myn = jnp.where(cid == 0, _C0N, _C1N)
    goff = jnp.where(cid == 0, 0, _C0N)
    tot = 4 * myn

    @pl.when(cid == 0)
    def _():
        pltpu.sync_copy(idx_hbm.at[:, sid], idx_v.at[0])
    pltpu.sync_copy(idx_hbm.at[:, 16 + sid], idx_v.at[1])

    bufs = (b0, b1, b2, b3)
    gsems = (g0, g1, g2, g3)
    wsems = (w0, w1, w2, w3)

    def coords(i):
        k = i // myn
        g = i - k * myn + goff
        sel = (g >= _NCHUNK).astype(jnp.int32)
        cc = g - sel * _NCHUNK
        base = (sid + 16 * sel) * _PPW + cc * _CW
        return k, sel, cc, base

    def gather(i, b, gsem):
        k, sel, cc, _ = coords(i)
        return pltpu.make_async_copy(table.at[idx_v.at[sel, k, cc]], b, gsem)

    def write(i, b, wsem):
        k, _, _, base = coords(i)
        return pltpu.make_async_copy(
            b, out.at[k, pl.ds(base, _CW)], wsem)

    for b in range(_NSLOT):
        gather(b, bufs[b], gsems[b]).start()

    def body(j, carry):
        for b in range(_NSLOT):
            i = j * _NSLOT + b
            gather(i, bufs[b], gsems[b]).wait()
            write(i, bufs[b], wsems[b]).start()
            i2 = i + _NSLOT

            @pl.when(i2 < tot)
            def _():
                write(i, bufs[b], wsems[b]).wait()
                gather(i2, bufs[b], gsems[b]).start()
        return carry

    lax.fori_loop(0, myn, body, 0)
    for b in range(_NSLOT):
        write(tot - _NSLOT + b, bufs[b], wsems[b]).wait()


@functools.lru_cache(maxsize=1)
def _sc_gather_fn():
    mesh = plsc.VectorSubcoreMesh(core_axis_name="c", subcore_axis_name="s")
    return functools.partial(
        pl.kernel, mesh=mesh,
        out_type=jax.ShapeDtypeStruct((4, _NPID, 128), jnp.float32),
        scratch_types=[
            pltpu.VMEM((2, 4, _NCHUNK, _CW), jnp.int32),
            pltpu.VMEM((_CW, 128), jnp.float32),
            pltpu.VMEM((_CW, 128), jnp.float32),
            pltpu.VMEM((_CW, 128), jnp.float32),
            pltpu.VMEM((_CW, 128), jnp.float32),
            pltpu.SemaphoreType.DMA,
            pltpu.SemaphoreType.DMA,
            pltpu.SemaphoreType.DMA,
            pltpu.SemaphoreType.DMA,
            pltpu.SemaphoreType.DMA,
            pltpu.SemaphoreType.DMA,
            pltpu.SemaphoreType.DMA,
            pltpu.SemaphoreType.DMA,
        ],
    )(_sc_gather_body)


def _gather_corners(table, idx4):
    return _sc_gather_fn()(table, idx4)


def _combine_body(g_ref, w_ref, fp_ref):
    w = w_ref[0]
    acc = None
    for k in range(4):
        gk = g_ref[k]
        term = (gk[:, 0:64] * w[2 * k][:, None]
                + gk[:, 64:128] * w[2 * k + 1][:, None])
        acc = term if acc is None else acc + term
    fp_ref[...] = acc


def _combine(g, wgt4):
    rows = _NBLK * _P
    return pl.pallas_call(
        _combine_body,
        grid=(_N // _NBLK,),
        in_specs=[
            pl.BlockSpec((4, rows, 128), lambda i: (0, i, 0)),
            pl.BlockSpec((1, 8, rows), lambda i: (i, 0, 0)),
        ],
        out_specs=pl.BlockSpec((rows, 64), lambda i: (i, 0)),
        out_shape=jax.ShapeDtypeStruct((_NPTS, 64), jnp.float32),
    )(g, wgt4)


def _wcomb_body(fw_ref, pw_ref, out_ref):
    pw = pw_ref[...]
    for o0 in range(0, 256, 64):
        wc = jnp.dot(fw_ref[o0:o0 + 64, :], pw,
                     preferred_element_type=jnp.float32)
        t = wc.reshape(64, 64, _P).transpose(0, 2, 1)
        out_ref[o0:o0 + 64] = t


def _wcomb(fuse_w, poly_w):
    return pl.pallas_call(
        _wcomb_body,
        out_shape=jax.ShapeDtypeStruct((256, _P, 64), jnp.float32),
    )(fuse_w, poly_w)


def _tail_body(fp_ref, ip_ref, wc_ref, fb_ref, pi_ref, pc_ref):
    fp = fp_ref[...]
    offs = jax.lax.dot_general(fp, wc_ref[...], (((1,), (1,)), ((), ())),
                               preferred_element_type=jnp.float32)
    offs = offs + fb_ref[...]
    ip = ip_ref[...]
    pi_ref[...] = ip * _DOWN_SAMPLE
    pc_ref[...] = offs * (_COARSE_STRIDE * _DOWN_SAMPLE) + ip * _DOWN_SAMPLE


def kernel(ct_hm, wh, cnn_feature, ct_01, ct_ind, ct_img_idx,
           conv1_w, conv1_b, conv2_w, conv2_b, poly_w, fuse_w, fuse_b):
    B, _, H, W = ct_hm.shape
    mask = ct_01.reshape(-1)
    ind = jnp.where(mask, ct_ind.reshape(-1), 0).astype(jnp.int32)
    img = jnp.where(mask, ct_img_idx.reshape(-1), 0).astype(jnp.int32)
    N = mask.shape[0]
    ct_x = ind % W
    ct_y = ind // W
    ct_offset = _wh_gather(wh, ind, img).T.reshape(N, -1, 2)
    ct = jnp.stack([ct_x.astype(jnp.float32), ct_y.astype(jnp.float32)], axis=1)
    init_polys = ct_offset * _INIT_STRIDE + ct[:, None, :]

    feat = _fused_conv(cnn_feature, conv1_w, conv1_b, conv2_w, conv2_b)
    table = feat.reshape(B * H * W // 2, 128)

    points = jnp.concatenate([ct[:, None, :], init_polys], axis=1)
    px = points[..., 0].reshape(-1)
    py = points[..., 1].reshape(-1)
    imgf = jnp.repeat(img.astype(jnp.float32), _P)
    pad = _NPID - _NPTS
    pts3 = jnp.stack([
        jnp.pad(px, (0, pad), constant_values=-1000.0),
        jnp.pad(py, (0, pad), constant_values=-1000.0),
        jnp.pad(imgf, (0, pad)),
    ] + [jnp.zeros(_NPID, jnp.float32)] * 5)

    idx4, wgt4 = _prep(pts3)
    idx4 = idx4.reshape(4, _NW, _NCHUNK, _CW)
    g = _gather_corners(table, idx4)
    fp = _combine(g, wgt4).reshape(_N, _P * 64)
    wcp = _wcomb(fuse_w, poly_w).reshape(256, _P * 64)

    ip_flat = init_polys.reshape(N, _NUM_POINT * 2)
    pi, pc = pl.pallas_call(
        _tail_body,
        out_shape=(jax.ShapeDtypeStruct((N, _NUM_POINT * 2), jnp.float32),
                   jax.ShapeDtypeStruct((N, _NUM_POINT * 2), jnp.float32)),
    )(fp, ip_flat, wcp, fuse_b.reshape(1, -1))
    return (pi.reshape(N, _NUM_POINT, 2), pc.reshape(N, _NUM_POINT, 2))


# speedup vs baseline: 1.1064x; 1.1053x over previous
import functools

import jax
import jax.numpy as jnp
from jax import lax
from jax.experimental import pallas as pl
from jax.experimental.pallas import tpu as pltpu
from jax.experimental.pallas import tpu_sc as plsc

_NUM_POINT = 128
_INIT_STRIDE = 10.0
_COARSE_STRIDE = 4.0
_DOWN_SAMPLE = 4.0
_TH = 8

_N = 400
_P = 129
_NPTS = _N * _P
_NW = 32
_CW = 128
_NCHUNK = 13
_PPW = _CW * _NCHUNK
_NPID = _NW * _PPW
_NBLK = 16


def _conv_body(x_ref, w1_ref, b1_ref, w2_ref, b2_ref, out_ref, xt_s):
    w1 = w1_ref[...]
    w2 = w2_ref[...]
    b1 = b1_ref[...]
    b2 = b2_ref[...]
    xt_s[0:1, :, :] = jnp.zeros((1, 130, 64), jnp.float32)
    xt_s[129:130, :, :] = jnp.zeros((1, 130, 64), jnp.float32)
    xt_s[1:129, 0:1, :] = jnp.zeros((128, 1, 64), jnp.float32)
    xt_s[1:129, 129:130, :] = jnp.zeros((128, 1, 64), jnp.float32)
    for r0 in range(0, 128, 16):
        blk = x_ref[0][:, r0:r0 + 16, :].reshape(64, 16 * 128)
        xt_s[r0 + 1:r0 + 17, 1:129, :] = blk.T.reshape(16, 128, 64)
    for h0 in range(0, 128, _TH):
        pieces = []
        for dy in range(3):
            for dx in range(3):
                xs = xt_s[h0 + dy:h0 + dy + _TH, dx:dx + 128, :]
                pieces.append(xs.reshape(_TH * 128, 64))
        x9 = jnp.concatenate(pieces, axis=1)
        acc = jnp.dot(x9, w1, preferred_element_type=jnp.float32) + b1
        acc = jnp.maximum(acc, 0.0)
        z = jnp.dot(acc, w2, preferred_element_type=jnp.float32) + b2
        out_ref[0, h0:h0 + _TH, :, :] = z.reshape(_TH, 128, 64)


def _fused_conv(x_nchw, conv1_w, conv1_b, conv2_w, conv2_b):
    B = x_nchw.shape[0]
    w1 = conv1_w.transpose(2, 3, 1, 0).reshape(576, 256)
    w2 = conv2_w[:, :, 0, 0].T
    return pl.pallas_call(
        _conv_body,
        grid=(B,),
        in_specs=[
            pl.BlockSpec((1, 64, 128, 128), lambda b: (b, 0, 0, 0)),
            pl.BlockSpec((576, 256), lambda b: (0, 0)),
            pl.BlockSpec((1, 256), lambda b: (0, 0)),
            pl.BlockSpec((256, 64), lambda b: (0, 0)),
            pl.BlockSpec((1, 64), lambda b: (0, 0)),
        ],
        out_specs=pl.BlockSpec((1, 128, 128, 64), lambda b: (b, 0, 0, 0)),
        out_shape=jax.ShapeDtypeStruct((B, 128, 128, 64), jnp.float32),
        scratch_shapes=[pltpu.VMEM((130, 130, 64), jnp.float32)],
    )(x_nchw, w1, conv1_b.reshape(1, 256), w2, conv2_b.reshape(1, 64))


def _prep_body(pts_ref, idx_ref, wgt_ref):
    x = pts_ref[0:1, :] - 0.5
    y = pts_ref[1:2, :] - 0.5
    imgi = pts_ref[2:3, :].astype(jnp.int32)
    x0 = jnp.floor(x)
    y0 = jnp.floor(y)
    wx1 = x - x0
    wy1 = y - y0
    for k, (dy, dx) in enumerate(((0, 0), (0, 1), (1, 0), (1, 1))):
        xc = x0 + dx
        yc = y0 + dy
        valid = (xc >= 0.0) & (xc < 128.0) & (yc >= 0.0) & (yc < 128.0)
        xi = jnp.clip(xc, 0.0, 127.0).astype(jnp.int32)
        yi = jnp.clip(yc, 0.0, 127.0).astype(jnp.int32)
        flat = imgi * 16384 + yi * 128 + xi
        idx = lax.shift_right_logical(flat, 1)
        odd = (flat & 1).astype(jnp.float32)
        wk = ((wx1 if dx else 1.0 - wx1) * (wy1 if dy else 1.0 - wy1)
              * valid.astype(jnp.float32))
        idx_ref[k:k + 1, :] = idx
        rows = _NBLK * _P
        for j in range(_N // _NBLK):
            sl = slice(j * rows, (j + 1) * rows)
            wgt_ref[j, 2 * k:2 * k + 1, :] = (wk * (1.0 - odd))[0:1, sl]
            wgt_ref[j, 2 * k + 1:2 * k + 2, :] = (wk * odd)[0:1, sl]


def _prep(pts3):
    return pl.pallas_call(
        _prep_body,
        out_shape=(jax.ShapeDtypeStruct((4, _NPID), jnp.int32),
                   jax.ShapeDtypeStruct((_N // _NBLK, 8, _NBLK * _P),
                                        jnp.float32)),
    )(pts3)


_NSLOT = 4
_TOTCH = 4 * _NCHUNK


def _sc_gather_body(table, idx_hbm, out, idx_v,
                    b0, b1, b2, b3, g0, g1, g2, g3, w0, w1, w2, w3):
    wid = lax.axis_index("c") * 16 + lax.axis_index("s")
    base = wid * _PPW
    pltpu.sync_copy(idx_hbm.at[:, wid], idx_v)
    bufs = (b0, b1, b2, b3)
    gsems = (g0, g1, g2, g3)
    wsems = (w0, w1, w2, w3)

    def gather(i, b, gsem):
        k = i // _NCHUNK
        c = i - k * _NCHUNK
        return pltpu.make_async_copy(table.at[idx_v.at[k, c]], b, gsem)

    def write(i, b, wsem):
        k = i // _NCHUNK
        c = i - k * _NCHUNK
        return pltpu.make_async_copy(
            b, out.at[k, pl.ds(base + c * _CW, _CW)], wsem)

    for b in range(_NSLOT):
        gather(b, bufs[b], gsems[b]).start()

    def body(j, carry):
        for b in range(_NSLOT):
            i = j * _NSLOT + b
            gather(i, bufs[b], gsems[b]).wait()
            write(i, bufs[b], wsems[b]).start()
            i2 = i + _NSLOT

            @pl.when(i2 < _TOTCH)
            def _():
                write(i, bufs[b], wsems[b]).wait()
                gather(i2, bufs[b], gsems[b]).start()
        return carry

    lax.fori_loop(0, _TOTCH // _NSLOT, body, 0)
    for b in range(_NSLOT):
        write(_TOTCH - _NSLOT + b, bufs[b], wsems[b]).wait()


@functools.lru_cache(maxsize=1)
def _sc_gather_fn():
    mesh = plsc.VectorSubcoreMesh(core_axis_name="c", subcore_axis_name="s")
    return functools.partial(
        pl.kernel, mesh=mesh,
        out_type=jax.ShapeDtypeStruct((4, _NPID, 128), jnp.float32),
        scratch_types=[
            pltpu.VMEM((4, _NCHUNK, _CW), jnp.int32),
            pltpu.VMEM((_CW, 128), jnp.float32),
            pltpu.VMEM((_CW, 128), jnp.float32),
            pltpu.VMEM((_CW, 128), jnp.float32),
            pltpu.VMEM((_CW, 128), jnp.float32),
            pltpu.SemaphoreType.DMA,
            pltpu.SemaphoreType.DMA,
            pltpu.SemaphoreType.DMA,
            pltpu.SemaphoreType.DMA,
            pltpu.SemaphoreType.DMA,
            pltpu.SemaphoreType.DMA,
            pltpu.SemaphoreType.DMA,
            pltpu.SemaphoreType.DMA,
        ],
    )(_sc_gather_body)


def _gather_corners(table, idx4):
    return _sc_gather_fn()(table, idx4)


def _combine_body(g_ref, w_ref, fp_ref):
    w = w_ref[0]
    acc = None
    for k in range(4):
        gk = g_ref[k]
        term = (gk[:, 0:64] * w[2 * k][:, None]
                + gk[:, 64:128] * w[2 * k + 1][:, None])
        acc = term if acc is None else acc + term
    fp_ref[...] = acc


def _combine(g, wgt4):
    rows = _NBLK * _P
    return pl.pallas_call(
        _combine_body,
        grid=(_N // _NBLK,),
        in_specs=[
            pl.BlockSpec((4, rows, 128), lambda i: (0, i, 0)),
            pl.BlockSpec((1, 8, rows), lambda i: (i, 0, 0)),
        ],
        out_specs=pl.BlockSpec((rows, 64), lambda i: (i, 0)),
        out_shape=jax.ShapeDtypeStruct((_NPTS, 64), jnp.float32),
    )(g, wgt4)


def _wcomb_body(fw_ref, pw_ref, out_ref):
    pw = pw_ref[...]
    for o0 in range(0, 256, 64):
        wc = jnp.dot(fw_ref[o0:o0 + 64, :], pw,
                     preferred_element_type=jnp.float32)
        t = wc.reshape(64, 64, _P).transpose(0, 2, 1)
        out_ref[o0:o0 + 64] = t


def _wcomb(fuse_w, poly_w):
    return pl.pallas_call(
        _wcomb_body,
        out_shape=jax.ShapeDtypeStruct((256, _P, 64), jnp.float32),
    )(fuse_w, poly_w)


def _tail_body(fp_ref, ip_ref, wc_ref, fb_ref, pi_ref, pc_ref):
    fp = fp_ref[...]
    offs = jax.lax.dot_general(fp, wc_ref[...], (((1,), (1,)), ((), ())),
                               preferred_element_type=jnp.float32)
    offs = offs + fb_ref[...]
    ip = ip_ref[...]
    pi_ref[...] = ip * _DOWN_SAMPLE
    pc_ref[...] = offs * (_COARSE_STRIDE * _DOWN_SAMPLE) + ip * _DOWN_SAMPLE


def kernel(ct_hm, wh, cnn_feature, ct_01, ct_ind, ct_img_idx,
           conv1_w, conv1_b, conv2_w, conv2_b, poly_w, fuse_w, fuse_b):
    B, _, H, W = ct_hm.shape
    mask = ct_01.reshape(-1)
    ind = jnp.where(mask, ct_ind.reshape(-1), 0).astype(jnp.int32)
    img = jnp.where(mask, ct_img_idx.reshape(-1), 0).astype(jnp.int32)
    N = mask.shape[0]
    ct_x = ind % W
    ct_y = ind // W
    ct_offset = wh[img, :, ct_y, ct_x].reshape(N, -1, 2)
    ct = jnp.stack([ct_x.astype(jnp.float32), ct_y.astype(jnp.float32)], axis=1)
    init_polys = ct_offset * _INIT_STRIDE + ct[:, None, :]

    feat = _fused_conv(cnn_feature, conv1_w, conv1_b, conv2_w, conv2_b)
    table = feat.reshape(B * H * W // 2, 128)

    points = jnp.concatenate([ct[:, None, :], init_polys], axis=1)
    px = points[..., 0].reshape(-1)
    py = points[..., 1].reshape(-1)
    imgf = jnp.repeat(img.astype(jnp.float32), _P)
    pad = _NPID - _NPTS
    pts3 = jnp.stack([
        jnp.pad(px, (0, pad), constant_values=-1000.0),
        jnp.pad(py, (0, pad), constant_values=-1000.0),
        jnp.pad(imgf, (0, pad)),
    ] + [jnp.zeros(_NPID, jnp.float32)] * 5)

    idx4, wgt4 = _prep(pts3)
    idx4 = idx4.reshape(4, _NW, _NCHUNK, _CW)
    g = _gather_corners(table, idx4)
    fp = _combine(g, wgt4).reshape(_N, _P * 64)
    wcp = _wcomb(fuse_w, poly_w).reshape(256, _P * 64)

    ip_flat = init_polys.reshape(N, _NUM_POINT * 2)
    pi, pc = pl.pallas_call(
        _tail_body,
        out_shape=(jax.ShapeDtypeStruct((N, _NUM_POINT * 2), jnp.float32),
                   jax.ShapeDtypeStruct((N, _NUM_POINT * 2), jnp.float32)),
    )(fp, ip_flat, wcp, fuse_b.reshape(1, -1))
    return (pi.reshape(N, _NUM_POINT, 2), pc.reshape(N, _NUM_POINT, 2))


# speedup vs baseline: 1.3810x; 1.2482x over previous
import functools

import jax
import jax.numpy as jnp
from jax import lax
from jax.experimental import pallas as pl
from jax.experimental.pallas import tpu as pltpu
from jax.experimental.pallas import tpu_sc as plsc

_NUM_POINT = 128
_INIT_STRIDE = 10.0
_COARSE_STRIDE = 4.0
_DOWN_SAMPLE = 4.0
_TH = 8

_N = 400
_P = 129
_NPTS = _N * _P
_NW = 32
_CW = 128
_NCHUNK = 13
_PPW = _CW * _NCHUNK
_NPID = _NW * _PPW
_NBLK = 16


def _conv_body(x_ref, w1_ref, b1_ref, w2_ref, b2_ref, out_ref, xt_s):
    w1 = w1_ref[...]
    w2 = w2_ref[...]
    b1 = b1_ref[...]
    b2 = b2_ref[...]
    xt_s[0:1, :, :] = jnp.zeros((1, 130, 64), jnp.float32)
    xt_s[129:130, :, :] = jnp.zeros((1, 130, 64), jnp.float32)
    xt_s[1:129, 0:1, :] = jnp.zeros((128, 1, 64), jnp.float32)
    xt_s[1:129, 129:130, :] = jnp.zeros((128, 1, 64), jnp.float32)
    for r0 in range(0, 128, 16):
        blk = x_ref[0][:, r0:r0 + 16, :].reshape(64, 16 * 128)
        xt_s[r0 + 1:r0 + 17, 1:129, :] = blk.T.reshape(16, 128, 64)
    for h0 in range(0, 128, _TH):
        pieces = []
        for dy in range(3):
            for dx in range(3):
                xs = xt_s[h0 + dy:h0 + dy + _TH, dx:dx + 128, :]
                pieces.append(xs.reshape(_TH * 128, 64))
        x9 = jnp.concatenate(pieces, axis=1)
        acc = jnp.dot(x9, w1, preferred_element_type=jnp.float32) + b1
        acc = jnp.maximum(acc, 0.0)
        z = jnp.dot(acc, w2, preferred_element_type=jnp.float32) + b2
        out_ref[0, h0:h0 + _TH, :, :] = z.reshape(_TH, 128, 64)


def _fused_conv(x_nchw, conv1_w, conv1_b, conv2_w, conv2_b):
    B = x_nchw.shape[0]
    w1 = conv1_w.transpose(2, 3, 1, 0).reshape(576, 256)
    w2 = conv2_w[:, :, 0, 0].T
    return pl.pallas_call(
        _conv_body,
        grid=(B,),
        in_specs=[
            pl.BlockSpec((1, 64, 128, 128), lambda b: (b, 0, 0, 0)),
            pl.BlockSpec((576, 256), lambda b: (0, 0)),
            pl.BlockSpec((1, 256), lambda b: (0, 0)),
            pl.BlockSpec((256, 64), lambda b: (0, 0)),
            pl.BlockSpec((1, 64), lambda b: (0, 0)),
        ],
        out_specs=pl.BlockSpec((1, 128, 128, 64), lambda b: (b, 0, 0, 0)),
        out_shape=jax.ShapeDtypeStruct((B, 128, 128, 64), jnp.float32),
        scratch_shapes=[pltpu.VMEM((130, 130, 64), jnp.float32)],
    )(x_nchw, w1, conv1_b.reshape(1, 256), w2, conv2_b.reshape(1, 64))


def _prep_body(pts_ref, idx_ref, wgt_ref):
    x = pts_ref[0:1, :] - 0.5
    y = pts_ref[1:2, :] - 0.5
    imgi = pts_ref[2:3, :].astype(jnp.int32)
    x0 = jnp.floor(x)
    y0 = jnp.floor(y)
    wx1 = x - x0
    wy1 = y - y0
    for k, (dy, dx) in enumerate(((0, 0), (0, 1), (1, 0), (1, 1))):
        xc = x0 + dx
        yc = y0 + dy
        valid = (xc >= 0.0) & (xc < 128.0) & (yc >= 0.0) & (yc < 128.0)
        xi = jnp.clip(xc, 0.0, 127.0).astype(jnp.int32)
        yi = jnp.clip(yc, 0.0, 127.0).astype(jnp.int32)
        idx = imgi * 16384 + yi * 128 + xi
        wk = ((wx1 if dx else 1.0 - wx1) * (wy1 if dy else 1.0 - wy1)
              * valid.astype(jnp.float32))
        idx_ref[k:k + 1, :] = idx
        rows = _NBLK * _P
        for j in range(_N // _NBLK):
            sl = slice(j * rows, (j + 1) * rows)
            wgt_ref[j, k:k + 1, :] = wk[0:1, sl]
            wgt_ref[j, k + 4:k + 5, :] = jnp.zeros((1, rows), jnp.float32)


def _prep(pts3):
    return pl.pallas_call(
        _prep_body,
        out_shape=(jax.ShapeDtypeStruct((4, _NPID), jnp.int32),
                   jax.ShapeDtypeStruct((_N // _NBLK, 8, _NBLK * _P),
                                        jnp.float32)),
    )(pts3)


_NSLOT = 4
_TOTCH = 4 * _NCHUNK


def _sc_gather_body(table, idx_hbm, out, idx_v,
                    b0, b1, b2, b3, g0, g1, g2, g3, w0, w1, w2, w3):
    wid = lax.axis_index("c") * 16 + lax.axis_index("s")
    base = wid * _PPW
    pltpu.sync_copy(idx_hbm.at[:, wid], idx_v)
    bufs = (b0, b1, b2, b3)
    gsems = (g0, g1, g2, g3)
    wsems = (w0, w1, w2, w3)

    def gather(i, b, gsem):
        k = i // _NCHUNK
        c = i - k * _NCHUNK
        return pltpu.make_async_copy(table.at[idx_v.at[k, c]], b, gsem)

    def write(i, b, wsem):
        k = i // _NCHUNK
        c = i - k * _NCHUNK
        return pltpu.make_async_copy(
            b, out.at[k, pl.ds(base + c * _CW, _CW)], wsem)

    for b in range(_NSLOT):
        gather(b, bufs[b], gsems[b]).start()

    def body(j, carry):
        for b in range(_NSLOT):
            i = j * _NSLOT + b
            gather(i, bufs[b], gsems[b]).wait()
            write(i, bufs[b], wsems[b]).start()
            i2 = i + _NSLOT

            @pl.when(i2 < _TOTCH)
            def _():
                write(i, bufs[b], wsems[b]).wait()
                gather(i2, bufs[b], gsems[b]).start()
        return carry

    lax.fori_loop(0, _TOTCH // _NSLOT, body, 0)
    for b in range(_NSLOT):
        write(_TOTCH - _NSLOT + b, bufs[b], wsems[b]).wait()


@functools.lru_cache(maxsize=1)
def _sc_gather_fn():
    mesh = plsc.VectorSubcoreMesh(core_axis_name="c", subcore_axis_name="s")
    return functools.partial(
        pl.kernel, mesh=mesh,
        compiler_params=pltpu.CompilerParams(use_tc_tiling_on_sc=False),
        out_type=jax.ShapeDtypeStruct((4, _NPID, 64), jnp.float32),
        scratch_types=[
            pltpu.VMEM((4, _NCHUNK, _CW), jnp.int32),
            pltpu.VMEM((_CW, 64), jnp.float32),
            pltpu.VMEM((_CW, 64), jnp.float32),
            pltpu.VMEM((_CW, 64), jnp.float32),
            pltpu.VMEM((_CW, 64), jnp.float32),
            pltpu.SemaphoreType.DMA,
            pltpu.SemaphoreType.DMA,
            pltpu.SemaphoreType.DMA,
            pltpu.SemaphoreType.DMA,
            pltpu.SemaphoreType.DMA,
            pltpu.SemaphoreType.DMA,
            pltpu.SemaphoreType.DMA,
            pltpu.SemaphoreType.DMA,
        ],
    )(_sc_gather_body)


def _gather_corners(table, idx4):
    return _sc_gather_fn()(table, idx4)


def _combine_body(g_ref, w_ref, fp_ref):
    w = w_ref[0]
    acc = None
    for k in range(4):
        term = g_ref[k] * w[k][:, None]
        acc = term if acc is None else acc + term
    fp_ref[...] = acc


def _combine(g, wgt4):
    rows = _NBLK * _P
    return pl.pallas_call(
        _combine_body,
        grid=(_N // _NBLK,),
        in_specs=[
            pl.BlockSpec((4, rows, 64), lambda i: (0, i, 0)),
            pl.BlockSpec((1, 8, rows), lambda i: (i, 0, 0)),
        ],
        out_specs=pl.BlockSpec((rows, 64), lambda i: (i, 0)),
        out_shape=jax.ShapeDtypeStruct((_NPTS, 64), jnp.float32),
    )(g, wgt4)


def _wcomb_body(fw_ref, pw_ref, out_ref):
    pw = pw_ref[...]
    for o0 in range(0, 256, 64):
        wc = jnp.dot(fw_ref[o0:o0 + 64, :], pw,
                     preferred_element_type=jnp.float32)
        t = wc.reshape(64, 64, _P).transpose(0, 2, 1)
        out_ref[o0:o0 + 64] = t


def _wcomb(fuse_w, poly_w):
    return pl.pallas_call(
        _wcomb_body,
        out_shape=jax.ShapeDtypeStruct((256, _P, 64), jnp.float32),
    )(fuse_w, poly_w)


def _tail_body(fp_ref, ip_ref, wc_ref, fb_ref, pi_ref, pc_ref):
    fp = fp_ref[...]
    offs = jax.lax.dot_general(fp, wc_ref[...], (((1,), (1,)), ((), ())),
                               preferred_element_type=jnp.float32)
    offs = offs + fb_ref[...]
    ip = ip_ref[...]
    pi_ref[...] = ip * _DOWN_SAMPLE
    pc_ref[...] = offs * (_COARSE_STRIDE * _DOWN_SAMPLE) + ip * _DOWN_SAMPLE


def kernel(ct_hm, wh, cnn_feature, ct_01, ct_ind, ct_img_idx,
           conv1_w, conv1_b, conv2_w, conv2_b, poly_w, fuse_w, fuse_b):
    B, _, H, W = ct_hm.shape
    mask = ct_01.reshape(-1)
    ind = jnp.where(mask, ct_ind.reshape(-1), 0).astype(jnp.int32)
    img = jnp.where(mask, ct_img_idx.reshape(-1), 0).astype(jnp.int32)
    N = mask.shape[0]
    ct_x = ind % W
    ct_y = ind // W
    ct_offset = wh[img, :, ct_y, ct_x].reshape(N, -1, 2)
    ct = jnp.stack([ct_x.astype(jnp.float32), ct_y.astype(jnp.float32)], axis=1)
    init_polys = ct_offset * _INIT_STRIDE + ct[:, None, :]

    feat = _fused_conv(cnn_feature, conv1_w, conv1_b, conv2_w, conv2_b)
    table = feat.reshape(B * H * W, 64)

    points = jnp.concatenate([ct[:, None, :], init_polys], axis=1)
    px = points[..., 0].reshape(-1)
    py = points[..., 1].reshape(-1)
    imgf = jnp.repeat(img.astype(jnp.float32), _P)
    pad = _NPID - _NPTS
    pts3 = jnp.stack([
        jnp.pad(px, (0, pad), constant_values=-1000.0),
        jnp.pad(py, (0, pad), constant_values=-1000.0),
        jnp.pad(imgf, (0, pad)),
    ] + [jnp.zeros(_NPID, jnp.float32)] * 5)

    idx4, wgt4 = _prep(pts3)
    idx4 = idx4.reshape(4, _NW, _NCHUNK, _CW)
    g = _gather_corners(table, idx4)
    fp = _combine(g, wgt4).reshape(_N, _P * 64)
    wcp = _wcomb(fuse_w, poly_w).reshape(256, _P * 64)

    ip_flat = init_polys.reshape(N, _NUM_POINT * 2)
    pi, pc = pl.pallas_call(
        _tail_body,
        out_shape=(jax.ShapeDtypeStruct((N, _NUM_POINT * 2), jnp.float32),
                   jax.ShapeDtypeStruct((N, _NUM_POINT * 2), jnp.float32)),
    )(fp, ip_flat, wcp, fuse_b.reshape(1, -1))
    return (pi.reshape(N, _NUM_POINT, 2), pc.reshape(N, _NUM_POINT, 2))
